# baseline (speedup 1.0000x reference)
import functools

import jax
import jax.numpy as jnp
from jax.experimental import pallas as pl
from jax.experimental.pallas import tpu as pltpu


def _adapter_block(x_ref, w1_ref, b1_ref, w2_ref, b2_ref, ln2_g_ref,
                   ln2_b_ref, o_ref):
    x = x_ref[...]
    m = jnp.mean(x, axis=-1, keepdims=True)
    c = x - m
    v = jnp.mean(c * c, axis=-1, keepdims=True)
    h = c * jax.lax.rsqrt(v + 1e-5)
    h1 = jnp.dot(h.astype(jnp.bfloat16), w1_ref[...],
                 preferred_element_type=jnp.float32) + b1_ref[...]
    g = 0.5 * h1 * (1.0 + jax.lax.erf(h1 * 0.7071067811865476))
    h2 = jnp.dot(g.astype(jnp.bfloat16), w2_ref[...],
                 preferred_element_type=jnp.float32) + b2_ref[...]
    m2 = jnp.mean(h2, axis=-1, keepdims=True)
    c2 = h2 - m2
    v2 = jnp.mean(c2 * c2, axis=-1, keepdims=True)
    o_ref[...] = c2 * jax.lax.rsqrt(v2 + 1e-5) * ln2_g_ref[...] + ln2_b_ref[...]


@functools.partial(jax.jit, static_argnames=("tm",))
def _run(x2d, w1b, b1, w2b, b2, ln2_g, ln2_b, tm):
    n, d = x2d.shape
    h = w1b.shape[1]
    o = w2b.shape[1]
    grid = (n // tm,)
    const = lambda i: (0, 0)
    out = pl.pallas_call(
        _adapter_block,
        grid=grid,
        in_specs=[
            pl.BlockSpec((tm, d), lambda i: (i, 0)),
            pl.BlockSpec((d, h), const),
            pl.BlockSpec((1, h), const),
            pl.BlockSpec((h, o), const),
            pl.BlockSpec((1, o), const),
            pl.BlockSpec((1, o), const),
            pl.BlockSpec((1, o), const),
        ],
        out_specs=pl.BlockSpec((tm, o), lambda i: (i, 0)),
        out_shape=jax.ShapeDtypeStruct((n, o), jnp.float32),
        compiler_params=pltpu.CompilerParams(
            dimension_semantics=("arbitrary",),
        ),
    )(x2d, w1b, b1, w2b, b2, ln2_g, ln2_b)
    return out


def kernel(x, ln_g, ln_b, W1, b1, W2, b2, ln2_g, ln2_b):
    B, T, D = x.shape
    H = W1.shape[1]
    O = W2.shape[1]
    x2d = x.reshape(B * T, D)
    w1b = (ln_g[:, None] * W1).astype(jnp.bfloat16)
    b1f = (ln_b @ W1 + b1).reshape(1, H)
    out = _run(x2d, w1b, b1f,
               W2.astype(jnp.bfloat16), b2.reshape(1, O),
               ln2_g.reshape(1, O), ln2_b.reshape(1, O),
               tm=512)
    return out.reshape(B, T, O)


# speedup vs baseline: 1.0615x; 1.0615x over previous
import functools

import jax
import jax.numpy as jnp
from jax.experimental import pallas as pl
from jax.experimental.pallas import tpu as pltpu


def _adapter_block(x_ref, ln_g_ref, ln_b_ref, w1_ref, b1_ref, w2_ref,
                   b2_ref, ln2_g_ref, ln2_b_ref, o_ref):
    x = x_ref[...]
    m = jnp.mean(x, axis=-1, keepdims=True)
    ex2 = jnp.mean(x * x, axis=-1, keepdims=True)
    s = jax.lax.rsqrt(ex2 - m * m + 1e-5)
    h = (x - m) * (s * ln_g_ref[...]) + ln_b_ref[...]
    h1 = jnp.dot(h.astype(jnp.bfloat16), w1_ref[...],
                 preferred_element_type=jnp.float32) + b1_ref[...]
    g = 0.5 * h1 * (1.0 + jax.lax.erf(h1 * 0.7071067811865476))
    h2 = jnp.dot(g.astype(jnp.bfloat16), w2_ref[...],
                 preferred_element_type=jnp.float32) + b2_ref[...]
    m2 = jnp.mean(h2, axis=-1, keepdims=True)
    e2 = jnp.mean(h2 * h2, axis=-1, keepdims=True)
    s2 = jax.lax.rsqrt(e2 - m2 * m2 + 1e-5)
    o_ref[...] = (h2 - m2) * (s2 * ln2_g_ref[...]) + ln2_b_ref[...]


@functools.partial(jax.jit, static_argnames=("tm",))
def _run(x2d, ln_g, ln_b, w1b, b1, w2b, b2, ln2_g, ln2_b, tm):
    n, d = x2d.shape
    h = w1b.shape[1]
    o = w2b.shape[1]
    grid = (n // tm,)
    const = lambda i: (0, 0)
    out = pl.pallas_call(
        _adapter_block,
        grid=grid,
        in_specs=[
            pl.BlockSpec((tm, d), lambda i: (i, 0)),
            pl.BlockSpec((1, d), const),
            pl.BlockSpec((1, d), const),
            pl.BlockSpec((d, h), const),
            pl.BlockSpec((1, h), const),
            pl.BlockSpec((h, o), const),
            pl.BlockSpec((1, o), const),
            pl.BlockSpec((1, o), const),
            pl.BlockSpec((1, o), const),
        ],
        out_specs=pl.BlockSpec((tm, o), lambda i: (i, 0)),
        out_shape=jax.ShapeDtypeStruct((n, o), jnp.float32),
        compiler_params=pltpu.CompilerParams(
            dimension_semantics=("arbitrary",),
        ),
    )(x2d, ln_g, ln_b, w1b, b1, w2b, b2, ln2_g, ln2_b)
    return out


def kernel(x, ln_g, ln_b, W1, b1, W2, b2, ln2_g, ln2_b):
    B, T, D = x.shape
    H = W1.shape[1]
    O = W2.shape[1]
    x2d = x.reshape(B * T, D)
    out = _run(x2d,
               ln_g.reshape(1, D), ln_b.reshape(1, D),
               W1.astype(jnp.bfloat16), b1.reshape(1, H),
               W2.astype(jnp.bfloat16), b2.reshape(1, O),
               ln2_g.reshape(1, O), ln2_b.reshape(1, O),
               tm=512)
    return out.reshape(B, T, O)
